# baseline (speedup 1.0000x reference)
import functools

import jax
import jax.numpy as jnp
from jax import lax
from jax.experimental import pallas as pl
from jax.experimental.pallas import tpu as pltpu
from jax.experimental.pallas import tpu_sc as plsc

B = 4096
L = 200
D = 64
NB = B * L
SCALE = 8.0

_INFO = plsc.get_sparse_core_info()
NC = _INFO.num_cores
NS = _INFO.num_subcores
NW = NC * NS
BPW = NB // NW
C = 512
NCHUNK = BPW // C

_mesh = plsc.VectorSubcoreMesh(core_axis_name="c", subcore_axis_name="s")


@functools.partial(
    pl.kernel,
    mesh=_mesh,
    compiler_params=pltpu.CompilerParams(use_tc_tiling_on_sc=False),
    out_type=jax.ShapeDtypeStruct((NB, D), jnp.float32),
    scratch_types=[
        pltpu.VMEM((C,), jnp.int32),
        pltpu.VMEM((C, D), jnp.float32),
        pltpu.SemaphoreType.DMA,
    ],
)
def _emb(idx_hbm, table_hbm, out_hbm, idx_v, rows_v, sem):
    wid = lax.axis_index("s") * NC + lax.axis_index("c")
    base = wid * BPW

    def chunk(g, carry):
        off = base + g * C
        pltpu.sync_copy(idx_hbm.at[pl.ds(off, C)], idx_v)
        pltpu.async_copy(table_hbm.at[idx_v], rows_v, sem).wait()

        def row(r, carry2):
            for j in range(D // 16):
                sl = pl.ds(j * 16, 16)
                rows_v[r, sl] = rows_v[r, sl] * SCALE
            return carry2

        lax.fori_loop(0, C, row, 0, unroll=2)
        pltpu.sync_copy(rows_v, out_hbm.at[pl.ds(off, C)])
        return carry

    lax.fori_loop(0, NCHUNK, chunk, 0)


def kernel(x, table):
    idx = x.reshape(NB).astype(jnp.int32)
    out = _emb(idx, table)
    return out.reshape(B, L, D)


# speedup vs baseline: 1.0051x; 1.0051x over previous
import functools

import jax
import jax.numpy as jnp
from jax import lax
from jax.experimental import pallas as pl
from jax.experimental.pallas import tpu as pltpu
from jax.experimental.pallas import tpu_sc as plsc

B = 4096
L = 200
D = 64
NB = B * L
N_TOK = 1000000
WN = N_TOK // 2
SCALE = 8.0

_INFO = plsc.get_sparse_core_info()
NC = _INFO.num_cores
NS = _INFO.num_subcores
NW = NC * NS
BPW = NB // NW
C = 256
NCHUNK = BPW // C

_mesh = plsc.VectorSubcoreMesh(core_axis_name="c", subcore_axis_name="s")


@functools.partial(
    pl.kernel,
    mesh=_mesh,
    compiler_params=pltpu.CompilerParams(use_tc_tiling_on_sc=True),
    out_type=jax.ShapeDtypeStruct((NB, D), jnp.float32),
    scratch_types=[
        pltpu.VMEM((C,), jnp.int32),
        pltpu.VMEM((C,), jnp.float32),
        pltpu.VMEM((C, 2 * D), jnp.float32),
        pltpu.VMEM((C, D), jnp.float32),
        pltpu.SemaphoreType.DMA,
    ],
)
def _emb(idx_hbm, tw_hbm, out_hbm, widx_v, par_v, wide_v, out_v, sem):
    wid = lax.axis_index("s") * NC + lax.axis_index("c")
    base = wid * BPW

    def chunk(g, carry):
        off = base + g * C
        pltpu.sync_copy(idx_hbm.at[pl.ds(off, C)], widx_v)

        def vec(i, c):
            sl = pl.ds(i * 16, 16)
            v = widx_v[sl]
            par_v[sl] = (v & 1).astype(jnp.float32)
            widx_v[sl] = v >> 1
            return c

        lax.fori_loop(0, C // 16, vec, 0, unroll=4)
        pltpu.async_copy(tw_hbm.at[widx_v], wide_v, sem).wait()

        def grp(gg, c):
            t0 = gg * 16
            pv = par_v[pl.ds(t0, 16)]
            for k in range(16):
                t = t0 + k
                p = lax.gather(
                    pv, jnp.full((16, 1), k, jnp.int32),
                    lax.GatherDimensionNumbers(
                        offset_dims=(), collapsed_slice_dims=(0,),
                        start_index_map=(0,)),
                    slice_sizes=(1,),
                    mode=lax.GatherScatterMode.PROMISE_IN_BOUNDS)
                for j in range(D // 16):
                    lo = wide_v[t, pl.ds(j * 16, 16)]
                    hi = wide_v[t, pl.ds(D + j * 16, 16)]
                    out_v[t, pl.ds(j * 16, 16)] = (lo + (hi - lo) * p) * SCALE
            return c

        lax.fori_loop(0, C // 16, grp, 0)
        pltpu.sync_copy(out_v, out_hbm.at[pl.ds(off, C)])
        return carry

    lax.fori_loop(0, NCHUNK, chunk, 0)


def kernel(x, table):
    idx = x.reshape(NB).astype(jnp.int32)
    tw = table.reshape(WN, 2 * D)
    out = _emb(idx, tw)
    return out.reshape(B, L, D)
